# baseline (speedup 1.0000x reference)
import jax
import jax.numpy as jnp
from jax import lax
from jax.experimental import pallas as pl
from jax.experimental.pallas import tpu as pltpu
from jax.experimental.pallas import tpu_sc as plsc

H = 64
M = 8
V = 128
B = 128
L = 8192
NW = 32
ROWS_PER_W = B // NW
LANES = 16


def _sc_body(seq_hbm, e_hbm, w_hbm, wb_hbm, wp_hbm, cnt_hbm, srow_hbm,
             seq_v, wp_v, tab_v, hist_v, score_v, e_v, w_v, wb_v,
             isem0, isem1, osem0, osem1):
    wid = lax.axis_index("s") * 2 + lax.axis_index("c")
    lane_iota = lax.iota(jnp.int32, LANES)
    ones = jnp.ones((LANES,), jnp.float32)
    zeros = jnp.zeros((LANES,), jnp.float32)
    isems = [isem0, isem1]
    osems = [osem0, osem1]

    rows = [wid * ROWS_PER_W + k for k in range(ROWS_PER_W)]
    in_h = [None] * ROWS_PER_W
    out_h = [None] * ROWS_PER_W
    in_h[0] = pltpu.async_copy(seq_hbm.at[rows[0]], seq_v.at[0], isems[0])
    pltpu.sync_copy(e_hbm, e_v)
    pltpu.sync_copy(w_hbm, w_v)
    pltpu.sync_copy(wb_hbm, wb_v)
    wvecs = [w_v[pl.ds(c * LANES, LANES)] for c in range(H // LANES)]
    wb = wb_v[pl.ds(0, LANES)]
    for g in range(V // LANES):
        S = zeros
        for j in range(LANES):
            v = g * LANES + j
            t = e_v[v, pl.ds(0, LANES)] * wvecs[0]
            for c in range(1, H // LANES):
                t = t + e_v[v, pl.ds(c * LANES, LANES)] * wvecs[c]
            tot = jnp.sum(t, axis=0)
            S = S + tot * jnp.where(lane_iota == j, jnp.float32(1),
                                    jnp.float32(0))
        S = S + wb
        score_v[pl.ds(g * LANES, LANES)] = S
        tab_v[pl.ds(g * LANES, LANES)] = 1.0 / (1.0 + jnp.exp(-S))

    @pl.when(wid == 0)
    def _():
        pltpu.sync_copy(score_v, srow_hbm.at[0])

    for k in range(ROWS_PER_W):
        b = k % 2
        if k + 1 < ROWS_PER_W:
            in_h[k + 1] = pltpu.async_copy(seq_hbm.at[rows[k + 1]],
                                           seq_v.at[1 - b], isems[1 - b])
        in_h[k].wait()
        if k >= 2:
            out_h[k - 2].wait()

        for g in range(V // LANES):
            hist_v[pl.ds(g * LANES, LANES)] = zeros

        @plsc.parallel_loop(0, L // LANES, unroll=16)
        def tok_body(i):
            base = i * LANES
            idx = seq_v[b, pl.ds(base, LANES)]
            wp_v[b, pl.ds(base, LANES)] = plsc.load_gather(tab_v, [idx])
            plsc.addupdate_scatter(hist_v, [idx], ones)

        out_h[k] = pltpu.async_copy(wp_v.at[b], wp_hbm.at[rows[k]], osems[b])
        pltpu.sync_copy(hist_v, cnt_hbm.at[rows[k]])
    out_h[ROWS_PER_W - 2].wait()
    out_h[ROWS_PER_W - 1].wait()


_sc_gather_hist = pl.kernel(
    _sc_body,
    out_type=[jax.ShapeDtypeStruct((B, L), jnp.float32),
              jax.ShapeDtypeStruct((B, V), jnp.float32),
              jax.ShapeDtypeStruct((1, V), jnp.float32)],
    mesh=plsc.VectorSubcoreMesh(core_axis_name="c", subcore_axis_name="s"),
    compiler_params=pltpu.CompilerParams(needs_layout_passes=False),
    scratch_types=[
        pltpu.VMEM((2, L), jnp.int32),
        pltpu.VMEM((2, L), jnp.float32),
        pltpu.VMEM((V,), jnp.float32),
        pltpu.VMEM((V,), jnp.float32),
        pltpu.VMEM((V,), jnp.float32),
        pltpu.VMEM((V, H), jnp.float32),
        pltpu.VMEM((H,), jnp.float32),
        pltpu.VMEM((LANES,), jnp.float32),
        pltpu.SemaphoreType.DMA,
        pltpu.SemaphoreType.DMA,
        pltpu.SemaphoreType.DMA,
        pltpu.SemaphoreType.DMA,
    ],
)


def _combine_body(cnt_ref, sv_ref, e_ref, rg_ref, rgb_ref,
                  hw_ref, hb_ref, logits_ref, rp_ref):
    cnt = cnt_ref[...]
    sv = sv_ref[...]
    su = lax.dot_general(sv, jnp.ones((1, V), jnp.float32),
                         (((0,), (0,)), ((), ())),
                         preferred_element_type=jnp.float32)
    iu = lax.broadcasted_iota(jnp.int32, (V, V), 0)
    iv = lax.broadcasted_iota(jnp.int32, (V, V), 1)
    prec = jnp.where((su > sv) | ((su == sv) & (iu < iv)),
                     jnp.float32(1), jnp.float32(0))
    ca = jnp.dot(cnt, prec, preferred_element_type=jnp.float32)
    tot = ca + cnt
    e = e_ref[...]
    ctx = jnp.dot(cnt, e, preferred_element_type=jnp.float32) * jnp.float32(1.0 / L)
    rg = rg_ref[...]
    a = lax.dot_general(ctx, rg[:, :H], (((1,), (1,)), ((), ())),
                        preferred_element_type=jnp.float32)
    rgb = rgb_ref[0, 0]
    den = jnp.zeros((B, 1), jnp.float32)
    wsum = jnp.zeros((B, H), jnp.float32)
    rps = []
    for j in range(M):
        selj = jnp.where((ca <= j) & (j < tot), jnp.float32(1), jnp.float32(0))
        memj = jnp.dot(selj, e, preferred_element_type=jnp.float32)
        mj = lax.dot_general(memj, rg[:, H:], (((1,), (1,)), ((), ())),
                             preferred_element_type=jnp.float32)
        rpj = jax.nn.sigmoid(a + mj + rgb)
        rps.append(rpj)
        den = den + rpj
        wsum = wsum + rpj * memj
    retrieved = wsum / (den + jnp.float32(1e-8))
    logits_ref[...] = lax.dot_general(retrieved, hw_ref[...],
                                      (((1,), (1,)), ((), ())),
                                      preferred_element_type=jnp.float32) + hb_ref[...]
    rp_ref[...] = jnp.concatenate(rps, axis=1)


def kernel(seq, embed_table, wg_w, wg_b, rg_w, rg_b, head_w, head_b):
    embed_table = embed_table.astype(jnp.float32)
    wb_arr = jnp.broadcast_to(jnp.asarray(wg_b, jnp.float32).reshape(1), (LANES,))
    write_prob, counts, score_row = _sc_gather_hist(
        seq.astype(jnp.int32), embed_table, wg_w.astype(jnp.float32), wb_arr)
    logits, read_prob = pl.pallas_call(
        _combine_body,
        out_shape=[jax.ShapeDtypeStruct((B, V), jnp.float32),
                   jax.ShapeDtypeStruct((B, M), jnp.float32)],
    )(counts, score_row, embed_table,
      rg_w.reshape(1, 2 * H).astype(jnp.float32),
      jnp.asarray(rg_b, jnp.float32).reshape(1, 1),
      head_w.astype(jnp.float32), head_b.reshape(1, V).astype(jnp.float32))
    return (logits, write_prob, read_prob)


# speedup vs baseline: 1.1735x; 1.1735x over previous
import jax
import jax.numpy as jnp
from jax import lax
from jax.experimental import pallas as pl
from jax.experimental.pallas import tpu as pltpu
from jax.experimental.pallas import tpu_sc as plsc

H = 64
M = 8
V = 128
B = 128
L = 8192
NW = 32
ROWS_PER_W = B // NW
LANES = 16


def _gate_body(e_ref, w_ref, b_ref, score_ref, score_row_ref, wp_row_ref):
    s = jnp.sum(e_ref[...] * w_ref[...], axis=1, keepdims=True) + b_ref[0, 0]
    score_ref[...] = s
    s_row = s.reshape(1, V)
    score_row_ref[...] = s_row
    wp_row_ref[...] = jax.nn.sigmoid(s_row)


def _gate_table(embed_table, wg_w, wg_b):
    return pl.pallas_call(
        _gate_body,
        out_shape=[jax.ShapeDtypeStruct((V, 1), jnp.float32),
                   jax.ShapeDtypeStruct((1, V), jnp.float32),
                   jax.ShapeDtypeStruct((1, V), jnp.float32)],
    )(embed_table, wg_w.reshape(1, H), wg_b.reshape(1, 1))


def _sc_body(seq_hbm, tab_hbm, wp_hbm, cnt_hbm, seq_v, wp_v, tab_v, hist_v, cnt_v,
             isem0, isem1, osem0, osem1):
    wid = lax.axis_index("s") * 2 + lax.axis_index("c")
    lane_base = lax.iota(jnp.int32, LANES) * V
    ones = jnp.ones((LANES,), jnp.float32)
    zeros = jnp.zeros((LANES,), jnp.float32)
    isems = [isem0, isem1]
    osems = [osem0, osem1]

    pltpu.sync_copy(tab_hbm, tab_v)

    rows = [wid * ROWS_PER_W + k for k in range(ROWS_PER_W)]
    in_h = [None] * ROWS_PER_W
    out_h = [None] * ROWS_PER_W
    in_h[0] = pltpu.async_copy(seq_hbm.at[rows[0]], seq_v.at[0], isems[0])
    for k in range(ROWS_PER_W):
        b = k % 2
        if k + 1 < ROWS_PER_W:
            in_h[k + 1] = pltpu.async_copy(seq_hbm.at[rows[k + 1]],
                                           seq_v.at[1 - b], isems[1 - b])
        in_h[k].wait()
        if k >= 2:
            out_h[k - 2].wait()

        for g in range(V // LANES):
            hist_v[pl.ds(g * LANES, LANES)] = zeros

        @plsc.parallel_loop(0, L // LANES, unroll=16)
        def tok_body(i):
            base = i * LANES
            idx = seq_v[b, pl.ds(base, LANES)]
            wp_v[b, pl.ds(base, LANES)] = plsc.load_gather(tab_v, [idx])
            plsc.addupdate_scatter(hist_v, [idx], ones)

        out_h[k] = pltpu.async_copy(wp_v.at[b], wp_hbm.at[rows[k]], osems[b])
        pltpu.sync_copy(hist_v, cnt_hbm.at[rows[k]])
    out_h[ROWS_PER_W - 2].wait()
    out_h[ROWS_PER_W - 1].wait()


_sc_gather_hist = pl.kernel(
    _sc_body,
    out_type=[jax.ShapeDtypeStruct((B, L), jnp.float32),
              jax.ShapeDtypeStruct((B, V), jnp.float32)],
    mesh=plsc.VectorSubcoreMesh(core_axis_name="c", subcore_axis_name="s"),
    compiler_params=pltpu.CompilerParams(needs_layout_passes=False),
    scratch_types=[
        pltpu.VMEM((2, L), jnp.int32),
        pltpu.VMEM((2, L), jnp.float32),
        pltpu.VMEM((V,), jnp.float32),
        pltpu.VMEM((V,), jnp.float32),
        pltpu.VMEM((V,), jnp.float32),
        pltpu.SemaphoreType.DMA,
        pltpu.SemaphoreType.DMA,
        pltpu.SemaphoreType.DMA,
        pltpu.SemaphoreType.DMA,
    ],
)


def _combine_body(cnt_ref, su_ref, sv_ref, e_ref, rg_ref, rgb_ref,
                  hw_ref, hb_ref, logits_ref, rp_ref):
    cnt = cnt_ref[...]
    su = su_ref[...]
    sv = sv_ref[...]
    iu = lax.broadcasted_iota(jnp.int32, (V, V), 0)
    iv = lax.broadcasted_iota(jnp.int32, (V, V), 1)
    prec = jnp.where((su > sv) | ((su == sv) & (iu < iv)),
                     jnp.float32(1), jnp.float32(0))
    ca = jnp.dot(cnt, prec, preferred_element_type=jnp.float32)
    tot = ca + cnt
    e = e_ref[...]
    ctx = jnp.dot(cnt, e, preferred_element_type=jnp.float32) * jnp.float32(1.0 / L)
    rg = rg_ref[...]
    a = lax.dot_general(ctx, rg[:, :H], (((1,), (1,)), ((), ())),
                        preferred_element_type=jnp.float32)
    rgb = rgb_ref[0, 0]
    den = jnp.zeros((B, 1), jnp.float32)
    wsum = jnp.zeros((B, H), jnp.float32)
    rps = []
    for j in range(M):
        selj = jnp.where((ca <= j) & (j < tot), jnp.float32(1), jnp.float32(0))
        memj = jnp.dot(selj, e, preferred_element_type=jnp.float32)
        mj = lax.dot_general(memj, rg[:, H:], (((1,), (1,)), ((), ())),
                             preferred_element_type=jnp.float32)
        rpj = jax.nn.sigmoid(a + mj + rgb)
        rps.append(rpj)
        den = den + rpj
        wsum = wsum + rpj * memj
    retrieved = wsum / (den + jnp.float32(1e-8))
    logits_ref[...] = lax.dot_general(retrieved, hw_ref[...],
                                      (((1,), (1,)), ((), ())),
                                      preferred_element_type=jnp.float32) + hb_ref[...]
    rp_ref[...] = jnp.concatenate([r.reshape(1, B) for r in rps], axis=0)


def kernel(seq, embed_table, wg_w, wg_b, rg_w, rg_b, head_w, head_b):
    embed_table = embed_table.astype(jnp.float32)
    score, score_row, wp_row = _gate_table(embed_table, wg_w.astype(jnp.float32),
                                           jnp.asarray(wg_b, jnp.float32))
    write_prob, counts = _sc_gather_hist(seq.astype(jnp.int32), wp_row.reshape(V))
    logits, read_prob_t = pl.pallas_call(
        _combine_body,
        out_shape=[jax.ShapeDtypeStruct((B, V), jnp.float32),
                   jax.ShapeDtypeStruct((M, B), jnp.float32)],
    )(counts, score, score_row, embed_table,
      rg_w.reshape(1, 2 * H).astype(jnp.float32),
      jnp.asarray(rg_b, jnp.float32).reshape(1, 1),
      head_w.astype(jnp.float32), head_b.reshape(1, V).astype(jnp.float32))
    return (logits, write_prob, read_prob_t.T)
